# baseline (device time: 9714 ns/iter reference)
import jax
import jax.numpy as jnp
from jax import lax
from jax.experimental import pallas as pl
from jax.experimental.pallas import tpu as pltpu

N_DEV = 4
N_CHUNK = 4


def kernel(x):
    m_rows, n_cols = x.shape
    chunk = m_rows // N_CHUNK

    def body(x_hbm, out_hbm, xbuf, stats_ref, in_sems, out_sems,
             send_sems, recv_sems):
        my_pos = lax.axis_index("i")

        barrier_sem = pltpu.get_barrier_semaphore()
        for d in range(1, N_DEV):
            peer = lax.rem(my_pos + d, N_DEV)
            pl.semaphore_signal(
                barrier_sem, inc=1,
                device_id=(peer,), device_id_type=pl.DeviceIdType.MESH,
            )

        in_copies = []
        for c in range(N_CHUNK):
            cp = pltpu.make_async_copy(
                x_hbm.at[pl.ds(c * chunk, chunk), :],
                xbuf.at[pl.ds(c * chunk, chunk), :],
                in_sems.at[c],
            )
            cp.start()
            in_copies.append(cp)

        m_parts, s_parts = [], []
        for c in range(N_CHUNK):
            in_copies[c].wait()
            xc = xbuf[pl.ds(c * chunk, chunk), :]
            mc = jnp.max(xc, axis=1, keepdims=True)
            ec = jnp.exp(xc - mc)
            sc = jnp.sum(ec, axis=1, keepdims=True)
            xbuf[pl.ds(c * chunk, chunk), :] = ec
            m_parts.append(mc)
            s_parts.append(sc)
        m = jnp.concatenate(m_parts, axis=0)
        s = jnp.concatenate(s_parts, axis=0)
        stats_ref[my_pos] = jnp.transpose(
            jnp.concatenate([m, s], axis=1)
        )

        pl.semaphore_wait(barrier_sem, N_DEV - 1)

        rdmas = []
        for d in (2, 1, 3):
            peer = lax.rem(my_pos + d, N_DEV)
            rdma = pltpu.make_async_remote_copy(
                src_ref=stats_ref.at[my_pos],
                dst_ref=stats_ref.at[my_pos],
                send_sem=send_sems.at[d - 1],
                recv_sem=recv_sems.at[d - 1],
                device_id=(peer,),
                device_id_type=pl.DeviceIdType.MESH,
            )
            rdma.start()
            rdmas.append(rdma)
        for rdma in rdmas:
            rdma.wait_recv()

        ms = [stats_ref[j, 0:1, :] for j in range(N_DEV)]
        ss = [stats_ref[j, 1:2, :] for j in range(N_DEV)]
        gmax = ms[0]
        for j in range(1, N_DEV):
            gmax = jnp.maximum(gmax, ms[j])
        gsum = ss[0] * jnp.exp(ms[0] - gmax)
        for j in range(1, N_DEV):
            gsum = gsum + ss[j] * jnp.exp(ms[j] - gmax)
        my_m = stats_ref[my_pos, 0:1, :]
        scale = jnp.transpose(jnp.exp(my_m - gmax) / gsum)

        out_copies = []
        for c in range(N_CHUNK):
            rows = pl.ds(c * chunk, chunk)
            xbuf[rows, :] = xbuf[rows, :] * scale[c * chunk:(c + 1) * chunk, :]
            cp = pltpu.make_async_copy(
                xbuf.at[rows, :], out_hbm.at[rows, :], out_sems.at[c]
            )
            cp.start()
            out_copies.append(cp)
        for cp in out_copies:
            cp.wait()
        for rdma in rdmas:
            rdma.wait_send()

    return pl.pallas_call(
        body,
        out_shape=jax.ShapeDtypeStruct((m_rows, n_cols), jnp.float32),
        in_specs=[pl.BlockSpec(memory_space=pltpu.MemorySpace.HBM)],
        out_specs=pl.BlockSpec(memory_space=pltpu.MemorySpace.HBM),
        scratch_shapes=[
            pltpu.VMEM((m_rows, n_cols), jnp.float32),
            pltpu.VMEM((N_DEV, 2, m_rows), jnp.float32),
            pltpu.SemaphoreType.DMA((N_CHUNK,)),
            pltpu.SemaphoreType.DMA((N_CHUNK,)),
            pltpu.SemaphoreType.DMA((N_DEV - 1,)),
            pltpu.SemaphoreType.DMA((N_DEV - 1,)),
        ],
        compiler_params=pltpu.CompilerParams(collective_id=0),
    )(x)


# device time: 9235 ns/iter; 1.0519x vs baseline; 1.0519x over previous
import jax
import jax.numpy as jnp
from jax import lax
from jax.experimental import pallas as pl
from jax.experimental.pallas import tpu as pltpu

N_DEV = 4
N_CHUNK = 2


def kernel(x):
    m_rows, n_cols = x.shape
    chunk = m_rows // N_CHUNK

    def body(x_hbm, out_hbm, xbuf, stats_ref, in_sems, out_sems,
             send_sems, recv_sems):
        my_pos = lax.axis_index("i")

        barrier_sem = pltpu.get_barrier_semaphore()
        for d in range(1, N_DEV):
            peer = lax.rem(my_pos + d, N_DEV)
            pl.semaphore_signal(
                barrier_sem, inc=1,
                device_id=(peer,), device_id_type=pl.DeviceIdType.MESH,
            )

        in_copies = []
        for c in range(N_CHUNK):
            cp = pltpu.make_async_copy(
                x_hbm.at[pl.ds(c * chunk, chunk), :],
                xbuf.at[pl.ds(c * chunk, chunk), :],
                in_sems.at[c],
            )
            cp.start()
            in_copies.append(cp)

        m_parts, s_parts = [], []
        for c in range(N_CHUNK):
            in_copies[c].wait()
            xc = xbuf[pl.ds(c * chunk, chunk), :]
            mc = jnp.max(xc, axis=1, keepdims=True)
            ec = jnp.exp(xc - mc)
            sc = jnp.sum(ec, axis=1, keepdims=True)
            xbuf[pl.ds(c * chunk, chunk), :] = ec
            m_parts.append(mc)
            s_parts.append(sc)
        m = jnp.concatenate(m_parts, axis=0)
        s = jnp.concatenate(s_parts, axis=0)
        stats_ref[my_pos] = jnp.transpose(
            jnp.concatenate([m, s], axis=1)
        )

        pl.semaphore_wait(barrier_sem, N_DEV - 1)

        rdmas = []
        for d in (2, 1, 3):
            peer = lax.rem(my_pos + d, N_DEV)
            rdma = pltpu.make_async_remote_copy(
                src_ref=stats_ref.at[my_pos],
                dst_ref=stats_ref.at[my_pos],
                send_sem=send_sems.at[d - 1],
                recv_sem=recv_sems.at[d - 1],
                device_id=(peer,),
                device_id_type=pl.DeviceIdType.MESH,
            )
            rdma.start()
            rdmas.append(rdma)
        for rdma in rdmas:
            rdma.wait_recv()

        ms = [stats_ref[j, 0:1, :] for j in range(N_DEV)]
        ss = [stats_ref[j, 1:2, :] for j in range(N_DEV)]
        gmax = ms[0]
        for j in range(1, N_DEV):
            gmax = jnp.maximum(gmax, ms[j])
        gsum = ss[0] * jnp.exp(ms[0] - gmax)
        for j in range(1, N_DEV):
            gsum = gsum + ss[j] * jnp.exp(ms[j] - gmax)
        my_m = stats_ref[my_pos, 0:1, :]
        scale = jnp.transpose(jnp.exp(my_m - gmax) / gsum)

        out_copies = []
        for c in range(N_CHUNK):
            rows = pl.ds(c * chunk, chunk)
            xbuf[rows, :] = xbuf[rows, :] * scale[c * chunk:(c + 1) * chunk, :]
            cp = pltpu.make_async_copy(
                xbuf.at[rows, :], out_hbm.at[rows, :], out_sems.at[c]
            )
            cp.start()
            out_copies.append(cp)
        for cp in out_copies:
            cp.wait()
        for rdma in rdmas:
            rdma.wait_send()

    return pl.pallas_call(
        body,
        out_shape=jax.ShapeDtypeStruct((m_rows, n_cols), jnp.float32),
        in_specs=[pl.BlockSpec(memory_space=pltpu.MemorySpace.HBM)],
        out_specs=pl.BlockSpec(memory_space=pltpu.MemorySpace.HBM),
        scratch_shapes=[
            pltpu.VMEM((m_rows, n_cols), jnp.float32),
            pltpu.VMEM((N_DEV, 2, m_rows), jnp.float32),
            pltpu.SemaphoreType.DMA((N_CHUNK,)),
            pltpu.SemaphoreType.DMA((N_CHUNK,)),
            pltpu.SemaphoreType.DMA((N_DEV - 1,)),
            pltpu.SemaphoreType.DMA((N_DEV - 1,)),
        ],
        compiler_params=pltpu.CompilerParams(collective_id=0),
    )(x)


# device time: 8541 ns/iter; 1.1373x vs baseline; 1.0813x over previous
import jax
import jax.numpy as jnp
from jax import lax
from jax.experimental import pallas as pl
from jax.experimental.pallas import tpu as pltpu

N_DEV = 4


def kernel(x):
    m_rows, n_cols = x.shape

    def body(x_ref, out_ref, stats_ref, send_sems, recv_sems):
        my_pos = lax.axis_index("i")

        barrier_sem = pltpu.get_barrier_semaphore()
        for d in range(1, N_DEV):
            peer = lax.rem(my_pos + d, N_DEV)
            pl.semaphore_signal(
                barrier_sem, inc=1,
                device_id=(peer,), device_id_type=pl.DeviceIdType.MESH,
            )

        xv = x_ref[:, :]
        m = jnp.max(xv, axis=1, keepdims=True)
        e = jnp.exp(xv - m)
        s = jnp.sum(e, axis=1, keepdims=True)
        stats_ref[my_pos] = jnp.transpose(
            jnp.concatenate([m, s], axis=1)
        )

        pl.semaphore_wait(barrier_sem, N_DEV - 1)

        rdmas = []
        for d in (2, 1, 3):
            peer = lax.rem(my_pos + d, N_DEV)
            rdma = pltpu.make_async_remote_copy(
                src_ref=stats_ref.at[my_pos],
                dst_ref=stats_ref.at[my_pos],
                send_sem=send_sems.at[d - 1],
                recv_sem=recv_sems.at[d - 1],
                device_id=(peer,),
                device_id_type=pl.DeviceIdType.MESH,
            )
            rdma.start()
            rdmas.append(rdma)
        for rdma in rdmas:
            rdma.wait_recv()
        for rdma in rdmas:
            rdma.wait_send()

        ms = [stats_ref[j, 0:1, :] for j in range(N_DEV)]
        ss = [stats_ref[j, 1:2, :] for j in range(N_DEV)]
        gmax = ms[0]
        for j in range(1, N_DEV):
            gmax = jnp.maximum(gmax, ms[j])
        gsum = ss[0] * jnp.exp(ms[0] - gmax)
        for j in range(1, N_DEV):
            gsum = gsum + ss[j] * jnp.exp(ms[j] - gmax)
        my_m = stats_ref[my_pos, 0:1, :]
        scale = jnp.exp(my_m - gmax) / gsum
        out_ref[:, :] = e * jnp.transpose(scale)

    return pl.pallas_call(
        body,
        out_shape=jax.ShapeDtypeStruct((m_rows, n_cols), jnp.float32),
        in_specs=[pl.BlockSpec(memory_space=pltpu.VMEM)],
        out_specs=pl.BlockSpec(memory_space=pltpu.VMEM),
        scratch_shapes=[
            pltpu.VMEM((N_DEV, 2, m_rows), jnp.float32),
            pltpu.SemaphoreType.DMA((N_DEV - 1,)),
            pltpu.SemaphoreType.DMA((N_DEV - 1,)),
        ],
        compiler_params=pltpu.CompilerParams(collective_id=0),
    )(x)
